# baseline (speedup 1.0000x reference)
import functools

import jax
import jax.numpy as jnp
from jax import lax
from jax.experimental import pallas as pl
from jax.experimental.pallas import tpu as pltpu
from jax.experimental.pallas import tpu_sc as plsc

NC = 2
NS = 16
NW = NC * NS
L = 16
CHUNK = 128


def _mirt_body(nchunk, nvec, n_stu, n_exer,
               stu_hbm, exer_hbm, soa_hbm,
               out_hbm,
               stu_v, exer_v, i1_v, i2_v, i3_v, i4_v,
               t0_v, t1_v, a0_v, a1_v, b_v, out_v, sem):
    wid = lax.axis_index("s") * NC + lax.axis_index("c")

    pltpu.sync_copy(stu_hbm.at[wid], stu_v)
    pltpu.sync_copy(exer_hbm.at[wid], exer_v)

    off_t1 = n_stu
    off_a0 = 2 * n_stu
    off_a1 = 2 * n_stu + n_exer
    off_b = 2 * n_stu + 2 * n_exer

    def idx_body(j, carry):
        sl = pl.ds(j * L, L)
        s = stu_v[sl]
        e = exer_v[sl]
        i1_v[sl] = s + off_t1
        i2_v[sl] = e + off_a0
        i3_v[sl] = e + off_a1
        i4_v[sl] = e + off_b
        return carry

    lax.fori_loop(0, nvec, idx_body, 0)

    copies = []
    for c in range(nchunk):
        sl = pl.ds(c * CHUNK, CHUNK)
        copies.append(pltpu.async_copy(soa_hbm.at[stu_v.at[sl]], t0_v.at[sl], sem))
        copies.append(pltpu.async_copy(soa_hbm.at[i1_v.at[sl]], t1_v.at[sl], sem))
        copies.append(pltpu.async_copy(soa_hbm.at[i2_v.at[sl]], a0_v.at[sl], sem))
        copies.append(pltpu.async_copy(soa_hbm.at[i3_v.at[sl]], a1_v.at[sl], sem))
        copies.append(pltpu.async_copy(soa_hbm.at[i4_v.at[sl]], b_v.at[sl], sem))
    for cp in copies:
        cp.wait()

    def vec_body(j, carry):
        sl = pl.ds(j * L, L)
        t0 = t0_v[sl]
        t1 = t1_v[sl]
        a0 = a0_v[sl]
        a1 = a1_v[sl]
        b = b_v[sl]
        pred = a0 * (t0 - b) + a1 * (t1 - b)
        out_v[sl] = 1.0 / (1.0 + jnp.exp(-pred))
        return carry

    lax.fori_loop(0, nvec, vec_body, 0)

    pltpu.sync_copy(out_v, out_hbm.at[wid])


def _build(batch, n_stu, n_exer):
    bpw = batch // NW
    nchunk = bpw // CHUNK
    nvec = bpw // L
    mesh = plsc.VectorSubcoreMesh(core_axis_name="c", subcore_axis_name="s")
    idx = pltpu.VMEM((bpw,), jnp.int32)
    val = pltpu.VMEM((bpw,), jnp.float32)
    return functools.partial(
        pl.kernel,
        out_type=jax.ShapeDtypeStruct((NW, bpw), jnp.float32),
        mesh=mesh,
        scratch_types=[idx, idx, idx, idx, idx, idx,
                       val, val, val, val, val, val,
                       pltpu.SemaphoreType.DMA],
    )(functools.partial(_mirt_body, nchunk, nvec, n_stu, n_exer))


def kernel(stu_id, exer_id, theta_table, alpha_table, beta_table):
    batch = stu_id.shape[0]
    bpw = batch // NW
    stu = stu_id.astype(jnp.int32).reshape(NW, bpw)
    exer = exer_id.astype(jnp.int32).reshape(NW, bpw)
    soa = jnp.concatenate([
        theta_table[:, 0], theta_table[:, 1],
        alpha_table[:, 0], alpha_table[:, 1],
        beta_table[:, 0],
    ])
    n_stu = theta_table.shape[0]
    n_exer = alpha_table.shape[0]
    out = _build(batch, n_stu, n_exer)(stu, exer, soa)
    return out.reshape(batch)


# speedup vs baseline: 1.8838x; 1.8838x over previous
import functools

import jax
import jax.numpy as jnp
from jax import lax
from jax.experimental import pallas as pl
from jax.experimental.pallas import tpu as pltpu
from jax.experimental.pallas import tpu_sc as plsc

NC = 2
NS = 16
NW = NC * NS
L = 16
CHUNK = 128


def _mirt_body(bpw, nchunk, nvec,
               stu_hbm, exer_hbm, t0_hbm, t1_hbm, a0_hbm, a1_hbm, b_hbm,
               out_hbm,
               stu_v, exer_v, t0_v, t1_v, a0_v, a1_v, b_v, out_v, sem):
    wid = lax.axis_index("s") * NC + lax.axis_index("c")
    base = wid * bpw

    pltpu.sync_copy(stu_hbm.at[pl.ds(base, bpw)], stu_v)
    pltpu.sync_copy(exer_hbm.at[pl.ds(base, bpw)], exer_v)

    copies = []
    for c in range(nchunk):
        sl = pl.ds(c * CHUNK, CHUNK)
        copies.append(pltpu.async_copy(t0_hbm.at[stu_v.at[sl]], t0_v.at[sl], sem))
        copies.append(pltpu.async_copy(t1_hbm.at[stu_v.at[sl]], t1_v.at[sl], sem))
        copies.append(pltpu.async_copy(a0_hbm.at[exer_v.at[sl]], a0_v.at[sl], sem))
        copies.append(pltpu.async_copy(a1_hbm.at[exer_v.at[sl]], a1_v.at[sl], sem))
        copies.append(pltpu.async_copy(b_hbm.at[exer_v.at[sl]], b_v.at[sl], sem))
    for cp in copies:
        cp.wait()

    def vec_body(j, carry):
        sl = pl.ds(j * L, L)
        t0 = t0_v[sl]
        t1 = t1_v[sl]
        a0 = a0_v[sl]
        a1 = a1_v[sl]
        b = b_v[sl]
        pred = a0 * (t0 - b) + a1 * (t1 - b)
        out_v[sl] = 1.0 / (1.0 + jnp.exp(-pred))
        return carry

    lax.fori_loop(0, nvec, vec_body, 0)

    pltpu.sync_copy(out_v, out_hbm.at[pl.ds(base, bpw)])


def _build(batch):
    bpw = batch // NW
    nchunk = bpw // CHUNK
    nvec = bpw // L
    mesh = plsc.VectorSubcoreMesh(core_axis_name="c", subcore_axis_name="s")
    idx = pltpu.VMEM((bpw,), jnp.int32)
    val = pltpu.VMEM((bpw,), jnp.float32)
    return functools.partial(
        pl.kernel,
        out_type=jax.ShapeDtypeStruct((batch,), jnp.float32),
        mesh=mesh,
        scratch_types=[idx, idx,
                       val, val, val, val, val, val,
                       pltpu.SemaphoreType.DMA],
    )(functools.partial(_mirt_body, bpw, nchunk, nvec))


def kernel(stu_id, exer_id, theta_table, alpha_table, beta_table):
    batch = stu_id.shape[0]
    stu = stu_id.astype(jnp.int32)
    exer = exer_id.astype(jnp.int32)
    t0 = theta_table[:, 0]
    t1 = theta_table[:, 1]
    a0 = alpha_table[:, 0]
    a1 = alpha_table[:, 1]
    b = beta_table.reshape(-1)
    return _build(batch)(stu, exer, t0, t1, a0, a1, b)


# speedup vs baseline: 3.4781x; 1.8463x over previous
import functools

import jax
import jax.numpy as jnp
from jax import lax
from jax.experimental import pallas as pl
from jax.experimental.pallas import tpu as pltpu
from jax.experimental.pallas import tpu_sc as plsc

NC = 2
NS = 16
NW = NC * NS
L = 16
CHUNK = 128


def _mirt_body(bpw, nchunk, nvec, n_stu,
               stu_hbm, exer_hbm, th_hbm, a0_hbm, a1_hbm, b_hbm,
               out_hbm,
               stu_v, exer_v, i1_v, t0_v, t1_v, a0_v, a1_v, b_v, out_v, sem):
    wid = lax.axis_index("s") * NC + lax.axis_index("c")
    base = wid * bpw

    pltpu.sync_copy(stu_hbm.at[pl.ds(base, bpw)], stu_v)
    pltpu.sync_copy(exer_hbm.at[pl.ds(base, bpw)], exer_v)

    def idx_body(j, carry):
        sl = pl.ds(j * L, L)
        i1_v[sl] = stu_v[sl] + n_stu
        return carry

    lax.fori_loop(0, nvec, idx_body, 0)

    copies = []
    for c in range(nchunk):
        sl = pl.ds(c * CHUNK, CHUNK)
        copies.append(pltpu.async_copy(th_hbm.at[stu_v.at[sl]], t0_v.at[sl], sem))
        copies.append(pltpu.async_copy(th_hbm.at[i1_v.at[sl]], t1_v.at[sl], sem))
        copies.append(pltpu.async_copy(a0_hbm.at[exer_v.at[sl]], a0_v.at[sl], sem))
        copies.append(pltpu.async_copy(a1_hbm.at[exer_v.at[sl]], a1_v.at[sl], sem))
        copies.append(pltpu.async_copy(b_hbm.at[exer_v.at[sl]], b_v.at[sl], sem))
    for cp in copies:
        cp.wait()

    def vec_body(j, carry):
        sl = pl.ds(j * L, L)
        t0 = t0_v[sl]
        t1 = t1_v[sl]
        a0 = a0_v[sl]
        a1 = a1_v[sl]
        b = b_v[sl]
        pred = a0 * (t0 - b) + a1 * (t1 - b)
        out_v[sl] = 1.0 / (1.0 + jnp.exp(-pred))
        return carry

    lax.fori_loop(0, nvec, vec_body, 0)

    pltpu.sync_copy(out_v, out_hbm.at[pl.ds(base, bpw)])


def _build(batch, n_stu):
    bpw = batch // NW
    nchunk = bpw // CHUNK
    nvec = bpw // L
    mesh = plsc.VectorSubcoreMesh(core_axis_name="c", subcore_axis_name="s")
    idx = pltpu.VMEM((bpw,), jnp.int32)
    val = pltpu.VMEM((bpw,), jnp.float32)
    return functools.partial(
        pl.kernel,
        out_type=jax.ShapeDtypeStruct((batch,), jnp.float32),
        mesh=mesh,
        scratch_types=[idx, idx, idx,
                       val, val, val, val, val, val,
                       pltpu.SemaphoreType.DMA],
    )(functools.partial(_mirt_body, bpw, nchunk, nvec, n_stu))


def kernel(stu_id, exer_id, theta_table, alpha_table, beta_table):
    batch = stu_id.shape[0]
    stu = stu_id.astype(jnp.int32)
    exer = exer_id.astype(jnp.int32)
    th_soa = jnp.ravel(theta_table.T)
    a0 = alpha_table[:, 0]
    a1 = alpha_table[:, 1]
    b = beta_table.reshape(-1)
    return _build(batch, theta_table.shape[0])(stu, exer, th_soa, a0, a1, b)


# speedup vs baseline: 3.6404x; 1.0467x over previous
import functools

import jax
import jax.numpy as jnp
from jax import lax
from jax.experimental import pallas as pl
from jax.experimental.pallas import tpu as pltpu
from jax.experimental.pallas import tpu_sc as plsc

NC = 2
NS = 16
NW = NC * NS
L = 16
CHUNK = 128


def _mirt_body(bpw, nchunk, nvec, n_stu, n_exer,
               stu_hbm, exer_hbm, th_hbm, ax_hbm, b_hbm,
               out_hbm,
               stu_v, exer_v, i1_v, i2_v,
               t0_v, t1_v, a0_v, a1_v, b_v, out_v, sem):
    wid = lax.axis_index("s") * NC + lax.axis_index("c")
    base = wid * bpw

    pltpu.sync_copy(stu_hbm.at[pl.ds(base, bpw)], stu_v)
    pltpu.sync_copy(exer_hbm.at[pl.ds(base, bpw)], exer_v)

    def idx_body(j, carry):
        sl = pl.ds(j * L, L)
        i1_v[sl] = stu_v[sl] + n_stu
        i2_v[sl] = exer_v[sl] + n_exer
        return carry

    lax.fori_loop(0, nvec, idx_body, 0)

    copies = []
    for c in range(nchunk):
        sl = pl.ds(c * CHUNK, CHUNK)
        copies.append(pltpu.async_copy(th_hbm.at[stu_v.at[sl]], t0_v.at[sl], sem))
        copies.append(pltpu.async_copy(th_hbm.at[i1_v.at[sl]], t1_v.at[sl], sem))
        copies.append(pltpu.async_copy(ax_hbm.at[exer_v.at[sl]], a0_v.at[sl], sem))
        copies.append(pltpu.async_copy(ax_hbm.at[i2_v.at[sl]], a1_v.at[sl], sem))
        copies.append(pltpu.async_copy(b_hbm.at[exer_v.at[sl]], b_v.at[sl], sem))
    for cp in copies:
        cp.wait()

    def vec_body(j, carry):
        sl = pl.ds(j * L, L)
        t0 = t0_v[sl]
        t1 = t1_v[sl]
        a0 = a0_v[sl]
        a1 = a1_v[sl]
        b = b_v[sl]
        pred = a0 * (t0 - b) + a1 * (t1 - b)
        out_v[sl] = 1.0 / (1.0 + jnp.exp(-pred))
        return carry

    lax.fori_loop(0, nvec, vec_body, 0)

    pltpu.sync_copy(out_v, out_hbm.at[pl.ds(base, bpw)])


def _build(batch, n_stu, n_exer):
    bpw = batch // NW
    nchunk = bpw // CHUNK
    nvec = bpw // L
    mesh = plsc.VectorSubcoreMesh(core_axis_name="c", subcore_axis_name="s")
    idx = pltpu.VMEM((bpw,), jnp.int32)
    val = pltpu.VMEM((bpw,), jnp.float32)
    return functools.partial(
        pl.kernel,
        out_type=jax.ShapeDtypeStruct((batch,), jnp.float32),
        mesh=mesh,
        scratch_types=[idx, idx, idx, idx,
                       val, val, val, val, val, val,
                       pltpu.SemaphoreType.DMA],
    )(functools.partial(_mirt_body, bpw, nchunk, nvec, n_stu, n_exer))


def kernel(stu_id, exer_id, theta_table, alpha_table, beta_table):
    batch = stu_id.shape[0]
    stu = stu_id.astype(jnp.int32)
    exer = exer_id.astype(jnp.int32)
    th_soa = jnp.ravel(theta_table.T)
    ax_soa = jnp.ravel(alpha_table.T)
    b = beta_table.reshape(-1)
    return _build(batch, theta_table.shape[0], alpha_table.shape[0])(
        stu, exer, th_soa, ax_soa, b)


# speedup vs baseline: 3.7054x; 1.0179x over previous
import functools

import jax
import jax.numpy as jnp
from jax import lax
from jax.experimental import pallas as pl
from jax.experimental.pallas import tpu as pltpu
from jax.experimental.pallas import tpu_sc as plsc

NC = 2
NS = 16
NW = NC * NS
L = 16
CHUNK = 128


def _mirt_body(bpw, nchunk, nvec, n_stu, n_exer,
               stu_hbm, exer_hbm, th_hbm, ax_hbm,
               out_hbm,
               stu_v, exer_v, i1_v, i2_v, i3_v,
               t0_v, t1_v, a0_v, a1_v, b_v, out_v, sem):
    wid = lax.axis_index("s") * NC + lax.axis_index("c")
    base = wid * bpw

    pltpu.sync_copy(stu_hbm.at[pl.ds(base, bpw)], stu_v)
    pltpu.sync_copy(exer_hbm.at[pl.ds(base, bpw)], exer_v)

    def idx_body(j, carry):
        sl = pl.ds(j * L, L)
        e = exer_v[sl]
        i1_v[sl] = stu_v[sl] + n_stu
        i2_v[sl] = e + n_exer
        i3_v[sl] = e + 2 * n_exer
        return carry

    lax.fori_loop(0, nvec, idx_body, 0)

    copies = []
    for c in range(nchunk):
        sl = pl.ds(c * CHUNK, CHUNK)
        copies.append(pltpu.async_copy(th_hbm.at[stu_v.at[sl]], t0_v.at[sl], sem))
        copies.append(pltpu.async_copy(th_hbm.at[i1_v.at[sl]], t1_v.at[sl], sem))
        copies.append(pltpu.async_copy(ax_hbm.at[exer_v.at[sl]], a0_v.at[sl], sem))
        copies.append(pltpu.async_copy(ax_hbm.at[i2_v.at[sl]], a1_v.at[sl], sem))
        copies.append(pltpu.async_copy(ax_hbm.at[i3_v.at[sl]], b_v.at[sl], sem))
    for cp in copies:
        cp.wait()

    def vec_body(j, carry):
        sl = pl.ds(j * L, L)
        t0 = t0_v[sl]
        t1 = t1_v[sl]
        a0 = a0_v[sl]
        a1 = a1_v[sl]
        b = b_v[sl]
        pred = a0 * (t0 - b) + a1 * (t1 - b)
        out_v[sl] = 1.0 / (1.0 + jnp.exp(-pred))
        return carry

    lax.fori_loop(0, nvec, vec_body, 0)

    pltpu.sync_copy(out_v, out_hbm.at[pl.ds(base, bpw)])


def _build(batch, n_stu, n_exer):
    bpw = batch // NW
    nchunk = bpw // CHUNK
    nvec = bpw // L
    mesh = plsc.VectorSubcoreMesh(core_axis_name="c", subcore_axis_name="s")
    idx = pltpu.VMEM((bpw,), jnp.int32)
    val = pltpu.VMEM((bpw,), jnp.float32)
    return functools.partial(
        pl.kernel,
        out_type=jax.ShapeDtypeStruct((batch,), jnp.float32),
        mesh=mesh,
        scratch_types=[idx, idx, idx, idx, idx,
                       val, val, val, val, val, val,
                       pltpu.SemaphoreType.DMA],
    )(functools.partial(_mirt_body, bpw, nchunk, nvec, n_stu, n_exer))


def kernel(stu_id, exer_id, theta_table, alpha_table, beta_table):
    batch = stu_id.shape[0]
    stu = stu_id.astype(jnp.int32)
    exer = exer_id.astype(jnp.int32)
    th_soa = jnp.ravel(theta_table.T)
    ax_soa = jnp.ravel(jnp.concatenate([alpha_table, beta_table], axis=1).T)
    return _build(batch, theta_table.shape[0], alpha_table.shape[0])(
        stu, exer, th_soa, ax_soa)


# speedup vs baseline: 3.7149x; 1.0025x over previous
import functools

import jax
import jax.numpy as jnp
from jax import lax
from jax.experimental import pallas as pl
from jax.experimental.pallas import tpu as pltpu
from jax.experimental.pallas import tpu_sc as plsc

NC = 2
NS = 16
NW = NC * NS
L = 16
CHUNK = 512


def _mirt_body(bpw, nchunk, nvec, n_stu, n_exer,
               stu_hbm, exer_hbm, th_hbm, ax_hbm,
               out_hbm,
               stu_v, exer_v, i1_v, i2_v, i3_v,
               t0_v, t1_v, a0_v, a1_v, b_v, out_v, sem):
    wid = lax.axis_index("s") * NC + lax.axis_index("c")
    base = wid * bpw

    pltpu.sync_copy(stu_hbm.at[pl.ds(base, bpw)], stu_v)
    pltpu.sync_copy(exer_hbm.at[pl.ds(base, bpw)], exer_v)

    def idx_body(j, carry):
        sl = pl.ds(j * L, L)
        e = exer_v[sl]
        i1_v[sl] = stu_v[sl] + n_stu
        i2_v[sl] = e + n_exer
        i3_v[sl] = e + 2 * n_exer
        return carry

    lax.fori_loop(0, nvec, idx_body, 0)

    copies = []
    for c in range(nchunk):
        sl = pl.ds(c * CHUNK, CHUNK)
        copies.append(pltpu.async_copy(th_hbm.at[stu_v.at[sl]], t0_v.at[sl], sem))
        copies.append(pltpu.async_copy(th_hbm.at[i1_v.at[sl]], t1_v.at[sl], sem))
        copies.append(pltpu.async_copy(ax_hbm.at[exer_v.at[sl]], a0_v.at[sl], sem))
        copies.append(pltpu.async_copy(ax_hbm.at[i2_v.at[sl]], a1_v.at[sl], sem))
        copies.append(pltpu.async_copy(ax_hbm.at[i3_v.at[sl]], b_v.at[sl], sem))
    for cp in copies:
        cp.wait()

    def vec_body(j, carry):
        sl = pl.ds(j * L, L)
        t0 = t0_v[sl]
        t1 = t1_v[sl]
        a0 = a0_v[sl]
        a1 = a1_v[sl]
        b = b_v[sl]
        pred = a0 * (t0 - b) + a1 * (t1 - b)
        out_v[sl] = 1.0 / (1.0 + jnp.exp(-pred))
        return carry

    lax.fori_loop(0, nvec, vec_body, 0)

    pltpu.sync_copy(out_v, out_hbm.at[pl.ds(base, bpw)])


def _build(batch, n_stu, n_exer):
    bpw = batch // NW
    nchunk = bpw // CHUNK
    nvec = bpw // L
    mesh = plsc.VectorSubcoreMesh(core_axis_name="c", subcore_axis_name="s")
    idx = pltpu.VMEM((bpw,), jnp.int32)
    val = pltpu.VMEM((bpw,), jnp.float32)
    return functools.partial(
        pl.kernel,
        out_type=jax.ShapeDtypeStruct((batch,), jnp.float32),
        mesh=mesh,
        scratch_types=[idx, idx, idx, idx, idx,
                       val, val, val, val, val, val,
                       pltpu.SemaphoreType.DMA],
    )(functools.partial(_mirt_body, bpw, nchunk, nvec, n_stu, n_exer))


def kernel(stu_id, exer_id, theta_table, alpha_table, beta_table):
    batch = stu_id.shape[0]
    stu = stu_id.astype(jnp.int32)
    exer = exer_id.astype(jnp.int32)
    th_soa = jnp.ravel(theta_table.T)
    ax_soa = jnp.ravel(jnp.concatenate([alpha_table, beta_table], axis=1).T)
    return _build(batch, theta_table.shape[0], alpha_table.shape[0])(
        stu, exer, th_soa, ax_soa)
